# baseline (speedup 1.0000x reference)
import functools

import jax
import jax.numpy as jnp
from jax import lax
from jax.experimental import pallas as pl
from jax.experimental.pallas import tpu as pltpu
from jax.experimental.pallas import tpu_sc as plsc

B = 8
N = 2048
S = 512
BS = B * S
RADII2 = (0.1 * 0.1, 0.2 * 0.2, 0.4 * 0.4)
KS_ = (16, 32, 64)
O1 = (32, 64, 64)


def _bf(x):
    return x.astype(jnp.bfloat16).astype(jnp.float32)


def _fps_body(px_ref, py_ref, pz_ref, far0_ref, cx_ref, cy_ref, cz_ref):
    px = px_ref[...]
    py = py_ref[...]
    pz = pz_ref[...]
    col = lax.broadcasted_iota(jnp.int32, (B, N), 1)
    colS = lax.broadcasted_iota(jnp.int32, (B, S), 1)
    zero = jnp.zeros((B, N), jnp.float32)

    def step(s, carry):
        dist, far, ax, ay, az = carry
        oh = col == far
        cxv = jnp.sum(jnp.where(oh, px, zero), axis=1, keepdims=True)
        cyv = jnp.sum(jnp.where(oh, py, zero), axis=1, keepdims=True)
        czv = jnp.sum(jnp.where(oh, pz, zero), axis=1, keepdims=True)
        ax = jnp.where(colS == s, cxv, ax)
        ay = jnp.where(colS == s, cyv, ay)
        az = jnp.where(colS == s, czv, az)
        dx = px - cxv
        dy = py - cyv
        dz = pz - czv
        t = dx * dx
        t = dy * dy + t
        t = dz * dz + t
        dist = jnp.minimum(dist, t)
        m = jnp.max(dist, axis=1, keepdims=True)
        far = jnp.min(jnp.where(dist == m, col, N), axis=1, keepdims=True)
        return dist, far, ax, ay, az

    dist0 = jnp.full((B, N), 1e10, jnp.float32)
    accS = jnp.zeros((B, S), jnp.float32)
    _, _, ax, ay, az = lax.fori_loop(0, S, step, (dist0, far0_ref[...], accS, accS, accS))
    cx_ref[...] = ax
    cy_ref[...] = ay
    cz_ref[...] = az


def _sqoff_body(fxt_ref, fyt_ref, fzt_ref, px_ref, py_ref, pz_ref,
                wp1_ref, wp2_ref, wp3_ref,
                sq_ref, off1_ref, off2_ref, off3_ref):
    fx = fxt_ref[0]
    fy = fyt_ref[0]
    fz = fzt_ref[0]
    px = px_ref[0]
    py = py_ref[0]
    pz = pz_ref[0]
    t = _bf(fx) * _bf(px)
    t = t + _bf(fy) * _bf(py)
    t = t + _bf(fz) * _bf(pz)
    s2 = fx * fx
    s2 = fy * fy + s2
    s2 = fz * fz + s2
    p2 = px * px
    p2 = py * py + p2
    p2 = pz * pz + p2
    d = -2.0 * t
    d = d + s2
    d = d + p2
    sq_ref[0] = d
    for wp_ref, off_ref in ((wp1_ref, off1_ref), (wp2_ref, off2_ref), (wp3_ref, off3_ref)):
        wp = wp_ref[...]
        off = _bf(fx) * _bf(wp[0:1, :])
        off = off + _bf(fy) * _bf(wp[1:2, :])
        off = off + _bf(fz) * _bf(wp[2:3, :])
        off_ref[0] = off


def _ypre_body(feat_ref, pxt_ref, pyt_ref, pzt_ref, w_ref, y1_ref, y2_ref, y3_ref):
    x = feat_ref[0]
    w = w_ref[...]
    acc = lax.dot_general(x.astype(jnp.bfloat16), w[:125, :].astype(jnp.bfloat16),
                          (((0,), (0,)), ((), ())),
                          preferred_element_type=jnp.float32)
    acc = acc + _bf(pxt_ref[0]) * _bf(w[125:126, :])
    acc = acc + _bf(pyt_ref[0]) * _bf(w[126:127, :])
    acc = acc + _bf(pzt_ref[0]) * _bf(w[127:128, :])
    y1_ref[0] = acc[:, 0:32]
    y2_ref[0] = acc[:, 32:96]
    y3_ref[0] = acc[:, 96:160]


def _ballq_gather(sq_hbm, yp1_hbm, yp2_hbm, yp3_hbm,
                  g1_hbm, g2_hbm, g3_hbm,
                  row_v, ib1_v, ib2_v, ib3_v, r1_v, r2_v, r3_v):
    wid = lax.axis_index("s") * 2 + lax.axis_index("c")
    nper = BS // 32
    base = wid * nper
    iota16 = lax.iota(jnp.int32, 16)

    def do_center(i, _):
        c = base + i
        b = c // S
        pbase = b * N
        pltpu.sync_copy(sq_hbm.at[pl.ds(c, 1)], row_v)
        pad0 = jnp.zeros((16,), jnp.int32) + (pbase + (N - 1))
        ib1_v[pl.ds(0, 16)] = pad0
        ib2_v[pl.ds(0, 16)] = pad0
        ib3_v[pl.ds(0, 16)] = pad0

        def chunk(j, cnts):
            v = row_v[0, pl.ds(j * 16, 16)]
            gidx = iota16 + (pbase + j * 16)
            new = []
            for si, (ib_v, kk, r2) in enumerate(((ib1_v, 16, RADII2[0]),
                                                 (ib2_v, 32, RADII2[1]),
                                                 (ib3_v, 64, RADII2[2]))):
                cnt = cnts[si]
                m = v <= r2
                cs = jnp.cumsum(m.astype(jnp.int32))
                pos = cnt + cs - 1
                wm = m & (pos < kk)
                plsc.store_scatter(ib_v, [pos], gidx, mask=wm)
                new.append(cnt + jnp.max(cs))
            return tuple(new)

        cnts = lax.fori_loop(0, N // 16, chunk, (jnp.int32(0), jnp.int32(0), jnp.int32(0)))
        for si, (ib_v, kk) in enumerate(((ib1_v, 16), (ib2_v, 32), (ib3_v, 64))):
            cnt = cnts[si]
            fsplat = plsc.load_gather(ib_v, [jnp.zeros((16,), jnp.int32)])
            for jj in range(kk // 16):
                lane = iota16 + jj * 16
                cur = ib_v[pl.ds(jj * 16, 16)]
                ib_v[pl.ds(jj * 16, 16)] = jnp.where(lane < cnt, cur, fsplat)
        pltpu.sync_copy(yp1_hbm.at[ib1_v], r1_v)
        pltpu.sync_copy(yp2_hbm.at[ib2_v], r2_v)
        pltpu.sync_copy(yp3_hbm.at[ib3_v], r3_v)
        pltpu.sync_copy(r1_v, g1_hbm.at[pl.ds(c * 16, 16)])
        pltpu.sync_copy(r2_v, g2_hbm.at[pl.ds(c * 32, 32)])
        pltpu.sync_copy(r3_v, g3_hbm.at[pl.ds(c * 64, 64)])
        return 0

    lax.fori_loop(0, nper, do_center, 0)


def _stats1_body(g_ref, off_ref, out_ref):
    pid = pl.program_id(0)
    x = g_ref[0] - off_ref[...]
    ps = jnp.sum(x, axis=0, keepdims=True)
    psq = jnp.sum(x * x, axis=0, keepdims=True)

    @pl.when(pid == 0)
    def _():
        out_ref[...] = jnp.zeros_like(out_ref)

    out_ref[0:1, :] += ps
    out_ref[1:2, :] += psq


def _layer_body(x_ref, off_ref, a_ref, c_ref, w_ref, y_ref, st_ref, *, has_off):
    pid = pl.program_id(0)
    x = x_ref[0]
    if has_off:
        x = x - off_ref[...]
    h = jnp.maximum(x * a_ref[...] + c_ref[...], 0.0)
    y = lax.dot_general(h.astype(jnp.bfloat16), w_ref[...].astype(jnp.bfloat16),
                        (((1,), (0,)), ((), ())), preferred_element_type=jnp.float32)
    y_ref[0] = y

    @pl.when(pid == 0)
    def _():
        st_ref[...] = jnp.zeros_like(st_ref)

    st_ref[0:1, :] += jnp.sum(y, axis=0, keepdims=True)
    st_ref[1:2, :] += jnp.sum(y * y, axis=0, keepdims=True)


def _fin_body(y_ref, a_ref, c_ref, out_ref):
    pid = pl.program_id(0)
    z = jnp.maximum(y_ref[0] * a_ref[...] + c_ref[...], 0.0)

    @pl.when(pid == 0)
    def _():
        out_ref[...] = jnp.full_like(out_ref, -jnp.inf)

    out_ref[...] = jnp.maximum(out_ref[...], z)


def _tc_ballq_body(sq_ref, yp1_ref, yp2_ref, yp3_ref, g1_ref, g2_ref, g3_ref):
    sqb = sq_ref[0]
    SB = sqb.shape[0]
    col = lax.broadcasted_iota(jnp.int32, (SB, N), 1)
    for (r2, k, yp_ref, g_ref) in ((RADII2[0], 16, yp1_ref, g1_ref),
                                   (RADII2[1], 32, yp2_ref, g2_ref),
                                   (RADII2[2], 64, yp3_ref, g3_ref)):
        ypb = yp_ref[0].astype(jnp.bfloat16)
        m = sqb <= r2
        first = None
        for t in range(k):
            val = jnp.where(m, col, N)
            mn = jnp.min(val, axis=1, keepdims=True)
            if t == 0:
                first = jnp.where(mn == N, N - 1, mn)
            sel = jnp.where(mn == N, first, mn)
            oh = (col == sel).astype(jnp.bfloat16)
            row = lax.dot_general(oh, ypb, (((1,), (0,)), ((), ())),
                                  preferred_element_type=jnp.float32)
            g_ref[t] = row
            m = m & (col > mn)


def _run_sc(sq_flat, yp1f, yp2f, yp3f):
    f32 = jnp.float32
    SB = 128
    nsb = S // SB
    sq3 = sq_flat.reshape(B, S, N)
    return pl.pallas_call(
        _tc_ballq_body,
        grid=(B, nsb),
        in_specs=[
            pl.BlockSpec((1, SB, N), lambda b, s: (b, s, 0)),
            pl.BlockSpec((1, N, 32), lambda b, s: (b, 0, 0)),
            pl.BlockSpec((1, N, 64), lambda b, s: (b, 0, 0)),
            pl.BlockSpec((1, N, 64), lambda b, s: (b, 0, 0)),
        ],
        out_specs=[
            pl.BlockSpec((16, SB, 32), lambda b, s: (0, b * 4 + s, 0)),
            pl.BlockSpec((32, SB, 64), lambda b, s: (0, b * 4 + s, 0)),
            pl.BlockSpec((64, SB, 64), lambda b, s: (0, b * 4 + s, 0)),
        ],
        out_shape=[
            jax.ShapeDtypeStruct((16, BS, 32), f32),
            jax.ShapeDtypeStruct((32, BS, 64), f32),
            jax.ShapeDtypeStruct((64, BS, 64), f32),
        ],
    )(sq3, yp1f.reshape(B, N, 32), yp2f.reshape(B, N, 64), yp3f.reshape(B, N, 64))


def _affine(stats, n, gamma, beta):
    mean = stats[0] / n
    var = stats[1] / n - mean * mean
    a = gamma / jnp.sqrt(var + 1e-5)
    c = beta - a * mean
    return a[None, :], c[None, :]


def kernel(bpc, bpc_features, params):
    f32 = jnp.float32
    px, py, pz = bpc[:, 0, :], bpc[:, 1, :], bpc[:, 2, :]
    far0 = jax.random.randint(jax.random.key(1), (B,), 0, N).astype(jnp.int32).reshape(B, 1)

    cx, cy, cz = pl.pallas_call(
        _fps_body,
        out_shape=[jax.ShapeDtypeStruct((B, S), f32)] * 3,
    )(px, py, pz, far0)

    new_xyz = jnp.stack([cx, cy, cz], axis=1)

    w1s = [params[i][0][0] for i in range(3)]
    wcat = jnp.concatenate(w1s, axis=0).T
    wpos = [w1s[i][:, 125:].T for i in range(3)]

    bspec = lambda shp: pl.BlockSpec(shp, lambda b: (b,) + (0,) * (len(shp) - 1))
    sq, off1, off2, off3 = pl.pallas_call(
        _sqoff_body,
        grid=(B,),
        in_specs=[
            bspec((1, S, 1)), bspec((1, S, 1)), bspec((1, S, 1)),
            bspec((1, 1, N)), bspec((1, 1, N)), bspec((1, 1, N)),
            pl.BlockSpec((3, 32), lambda b: (0, 0)),
            pl.BlockSpec((3, 64), lambda b: (0, 0)),
            pl.BlockSpec((3, 64), lambda b: (0, 0)),
        ],
        out_specs=[bspec((1, S, N)), bspec((1, S, 32)), bspec((1, S, 64)), bspec((1, S, 64))],
        out_shape=[
            jax.ShapeDtypeStruct((B, S, N), f32),
            jax.ShapeDtypeStruct((B, S, 32), f32),
            jax.ShapeDtypeStruct((B, S, 64), f32),
            jax.ShapeDtypeStruct((B, S, 64), f32),
        ],
    )(cx[:, :, None], cy[:, :, None], cz[:, :, None],
      px[:, None, :], py[:, None, :], pz[:, None, :], wpos[0], wpos[1], wpos[2])

    pxt = px[:, :, None]
    pyt = py[:, :, None]
    pzt = pz[:, :, None]
    yp1, yp2, yp3 = pl.pallas_call(
        _ypre_body,
        grid=(B,),
        in_specs=[
            bspec((1, 125, N)),
            bspec((1, N, 1)), bspec((1, N, 1)), bspec((1, N, 1)),
            pl.BlockSpec((128, 160), lambda b: (0, 0)),
        ],
        out_specs=[bspec((1, N, 32)), bspec((1, N, 64)), bspec((1, N, 64))],
        out_shape=[
            jax.ShapeDtypeStruct((B, N, 32), f32),
            jax.ShapeDtypeStruct((B, N, 64), f32),
            jax.ShapeDtypeStruct((B, N, 64), f32),
        ],
    )(bpc_features, pxt, pyt, pzt, wcat)

    g1, g2, g3 = _run_sc(sq.reshape(BS, N),
                         yp1.reshape(B * N, 32), yp2.reshape(B * N, 64), yp3.reshape(B * N, 64))

    outs = []
    for si, (g, off, k, o1) in enumerate(((g1, off1, 16, 32), (g2, off2, 32, 64), (g3, off3, 64, 64))):
        layers = params[si]
        grid = (k,)
        offf = off.reshape(BS, o1)
        n_elems = jnp.float32(BS * k)

        stats1 = pl.pallas_call(
            _stats1_body,
            grid=grid,
            in_specs=[
                pl.BlockSpec((1, BS, o1), lambda i: (i, 0, 0)),
                pl.BlockSpec((BS, o1), lambda i: (0, 0)),
            ],
            out_specs=pl.BlockSpec((8, o1), lambda i: (0, 0)),
            out_shape=jax.ShapeDtypeStruct((8, o1), f32),
        )(g, offf)
        a1, c1 = _affine(stats1, n_elems, layers[0][2], layers[0][3])

        w2 = layers[1][0].T
        o2 = w2.shape[1]
        y2, stats2 = pl.pallas_call(
            functools.partial(_layer_body, has_off=True),
            grid=grid,
            in_specs=[
                pl.BlockSpec((1, BS, o1), lambda i: (i, 0, 0)),
                pl.BlockSpec((BS, o1), lambda i: (0, 0)),
                pl.BlockSpec((1, o1), lambda i: (0, 0)),
                pl.BlockSpec((1, o1), lambda i: (0, 0)),
                pl.BlockSpec((o1, o2), lambda i: (0, 0)),
            ],
            out_specs=[pl.BlockSpec((1, BS, o2), lambda i: (i, 0, 0)),
                       pl.BlockSpec((8, o2), lambda i: (0, 0))],
            out_shape=[jax.ShapeDtypeStruct((k, BS, o2), f32),
                       jax.ShapeDtypeStruct((8, o2), f32)],
        )(g, offf, a1, c1, w2)
        a2, c2 = _affine(stats2, n_elems, layers[1][2], layers[1][3])

        w3 = layers[2][0].T
        o3 = w3.shape[1]
        y3, stats3 = pl.pallas_call(
            functools.partial(_layer_body, has_off=False),
            grid=grid,
            in_specs=[
                pl.BlockSpec((1, BS, o2), lambda i: (i, 0, 0)),
                pl.BlockSpec((BS, o1), lambda i: (0, 0)),
                pl.BlockSpec((1, o2), lambda i: (0, 0)),
                pl.BlockSpec((1, o2), lambda i: (0, 0)),
                pl.BlockSpec((o2, o3), lambda i: (0, 0)),
            ],
            out_specs=[pl.BlockSpec((1, BS, o3), lambda i: (i, 0, 0)),
                       pl.BlockSpec((8, o3), lambda i: (0, 0))],
            out_shape=[jax.ShapeDtypeStruct((k, BS, o3), f32),
                       jax.ShapeDtypeStruct((8, o3), f32)],
        )(y2, offf, a2, c2, w3)
        a3, c3 = _affine(stats3, n_elems, layers[2][2], layers[2][3])

        feat = pl.pallas_call(
            _fin_body,
            grid=grid,
            in_specs=[
                pl.BlockSpec((1, BS, o3), lambda i: (i, 0, 0)),
                pl.BlockSpec((1, o3), lambda i: (0, 0)),
                pl.BlockSpec((1, o3), lambda i: (0, 0)),
            ],
            out_specs=pl.BlockSpec((BS, o3), lambda i: (0, 0)),
            out_shape=jax.ShapeDtypeStruct((BS, o3), f32),
        )(y3, a3, c3)
        outs.append(feat.reshape(B, S, o3).transpose(0, 2, 1))

    new_features = jnp.concatenate(outs, axis=1)
    return new_xyz, new_features
